# baseline (speedup 1.0000x reference)
import functools

import jax
import jax.numpy as jnp
from jax import lax
from jax.experimental import pallas as pl
from jax.experimental.pallas import tpu as pltpu
from jax.experimental.pallas import tpu_sc as plsc

N_ROWS = 16384
N_COLS = 1000

TC_ROWS = 0
SC_ROWS = N_ROWS - TC_ROWS

TC_BLOCK_ROWS = 2048

NC, NS, L = 2, 16, 16
NW = NC * NS
SC_ROW0 = TC_ROWS
RPW = SC_ROWS // NW
G = RPW // L
UNROLL = 4


def _tc_kernel(x_ref, t_ref, out_ref, acc_ref):
    i = pl.program_id(0)

    @pl.when(i == 0)
    def _init():
        acc_ref[0] = 0.0
        acc_ref[1] = 0.0

    x = x_ref[...]
    m = jnp.max(x, axis=1, keepdims=True)
    e = jnp.exp(x - m)
    ones = jnp.ones((N_COLS, 128), jnp.float32)
    s = lax.dot_general(e, ones, (((1,), (0,)), ((), ())),
                        preferred_element_type=jnp.float32)[:, :1]
    conf = 1.0 / s[:, 0]

    idx = lax.broadcasted_iota(jnp.int32, x.shape, 1)
    pred = jnp.min(jnp.where(x == m, idx, N_COLS), axis=1)
    acc = (pred == t_ref[...]).astype(jnp.float32)

    acc_ref[0] += jnp.sum(acc)
    acc_ref[1] += jnp.sum(conf)

    @pl.when(i == pl.num_programs(0) - 1)
    def _fini():
        out_ref[...] = jnp.stack([acc_ref[0], acc_ref[1]]).reshape(1, 2)


def _tc_part(output, target):
    grid = TC_ROWS // TC_BLOCK_ROWS
    return pl.pallas_call(
        _tc_kernel,
        grid=(grid,),
        in_specs=[
            pl.BlockSpec((TC_BLOCK_ROWS, N_COLS), lambda i: (i, 0)),
            pl.BlockSpec((TC_BLOCK_ROWS,), lambda i: (i,)),
        ],
        out_specs=pl.BlockSpec((1, 2), lambda i: (0, 0)),
        out_shape=jax.ShapeDtypeStruct((1, 2), jnp.float32),
        scratch_shapes=[pltpu.SMEM((2,), jnp.float32)],
    )(output, target)


def _sc_body(x_hbm, t_hbm, acc_out, conf_out, buf0, buf1, tbuf, outv, sem0,
             sem1, tsem, osem):
    wid = lax.axis_index("s") * NC + lax.axis_index("c")
    row0 = SC_ROW0 + wid * RPW

    iota = lax.iota(jnp.int32, L)

    tcopy = pltpu.make_async_copy(t_hbm.at[pl.ds(row0, RPW)], tbuf, tsem)
    tcopy.start()

    bufs = (buf0, buf1)
    sems = (sem0, sem1)

    def start(g):
        cp = pltpu.make_async_copy(
            x_hbm.at[pl.ds(row0 + g * L, L)], bufs[g % 2], sems[g % 2])
        cp.start()
        return cp

    pending = start(0)
    tcopy.wait()

    conf_acc = jnp.zeros((L,), jnp.float32)
    acc_acc = jnp.zeros((L,), jnp.float32)

    for g in range(G):
        pending.wait()
        if g + 1 < G:
            pending = start(g + 1)
        cur = bufs[g % 2]

        def p1(k, ms):
            out = []
            for c in range(UNROLL):
                col = k * UNROLL + c
                colv = jnp.full((L,), col, jnp.int32)
                v = plsc.load_gather(cur, [iota, colv])
                out.append(jnp.maximum(ms[c], v))
            return tuple(out)

        ms = lax.fori_loop(0, N_COLS // UNROLL, p1,
                           tuple(jnp.full((L,), -jnp.inf, jnp.float32)
                                 for _ in range(UNROLL)))
        m = jnp.maximum(jnp.maximum(ms[0], ms[1]), jnp.maximum(ms[2], ms[3]))

        def p2(k, carry):
            ss, ii = carry
            ss, ii = list(ss), list(ii)
            for c in range(UNROLL):
                col = k * UNROLL + c
                colv = jnp.full((L,), col, jnp.int32)
                v = plsc.load_gather(cur, [iota, colv])
                ss[c] = ss[c] + jnp.exp(v - m)
                ii[c] = jnp.minimum(ii[c], jnp.where(v == m, colv, N_COLS))
            return tuple(ss), tuple(ii)

        (ss, ii) = lax.fori_loop(
            0, N_COLS // UNROLL, p2,
            (tuple(jnp.zeros((L,), jnp.float32) for _ in range(UNROLL)),
             tuple(jnp.full((L,), N_COLS, jnp.int32) for _ in range(UNROLL))))
        s = (ss[0] + ss[1]) + (ss[2] + ss[3])
        pred = jnp.minimum(jnp.minimum(ii[0], ii[1]),
                           jnp.minimum(ii[2], ii[3]))

        t_vec = tbuf[pl.ds(g * L, L)]
        conf_acc = conf_acc + 1.0 / s
        acc_acc = acc_acc + jnp.where(pred == t_vec, 1.0, 0.0)

    outv[pl.ds(0, L)] = acc_acc
    outv[pl.ds(L, L)] = conf_acc
    pltpu.make_async_copy(outv.at[pl.ds(0, L)], acc_out.at[wid], osem).start()
    pltpu.make_async_copy(outv.at[pl.ds(0, L)], acc_out.at[wid], osem).wait()
    pltpu.make_async_copy(outv.at[pl.ds(L, L)], conf_out.at[wid], osem).start()
    pltpu.make_async_copy(outv.at[pl.ds(L, L)], conf_out.at[wid], osem).wait()


def _sc_part(output, target):
    mesh = plsc.VectorSubcoreMesh(core_axis_name="c", subcore_axis_name="s")
    f = pl.kernel(
        _sc_body,
        out_type=[
            jax.ShapeDtypeStruct((NW, L), jnp.float32),
            jax.ShapeDtypeStruct((NW, L), jnp.float32),
        ],
        mesh=mesh,
        scratch_types=[
            pltpu.VMEM((L, N_COLS), jnp.float32),
            pltpu.VMEM((L, N_COLS), jnp.float32),
            pltpu.VMEM((RPW,), jnp.int32),
            pltpu.VMEM((2 * L,), jnp.float32),
            pltpu.SemaphoreType.DMA,
            pltpu.SemaphoreType.DMA,
            pltpu.SemaphoreType.DMA,
            pltpu.SemaphoreType.DMA,
        ],
        compiler_params=pltpu.CompilerParams(use_tc_tiling_on_sc=False, needs_layout_passes=False),
    )
    return f(output, target)


@jax.jit
def _ece(output, target):
    target = target.astype(jnp.int32)
    acc_sum = jnp.float32(0)
    conf_sum = jnp.float32(0)
    if TC_ROWS > 0:
        tc = _tc_part(output, target)
        acc_sum += tc[0, 0]
        conf_sum += tc[0, 1]
    if SC_ROWS > 0:
        acc_p, conf_p = _sc_part(output, target)
        acc_sum += jnp.sum(acc_p)
        conf_sum += jnp.sum(conf_p)
    return jnp.abs(acc_sum - conf_sum) / N_ROWS


def kernel(output, target):
    return _ece(output, target)


# speedup vs baseline: 1.4057x; 1.4057x over previous
import functools

import jax
import jax.numpy as jnp
from jax import lax
from jax.experimental import pallas as pl
from jax.experimental.pallas import tpu as pltpu
from jax.experimental.pallas import tpu_sc as plsc

N_ROWS = 16384
N_COLS = 1000

TC_ROWS = 12288
SC_ROWS = N_ROWS - TC_ROWS

TC_BLOCK_ROWS = 2048

NC, NS, L = 2, 16, 16
NW = NC * NS
SC_ROW0 = TC_ROWS
RPW = SC_ROWS // NW
G = RPW // L
UNROLL = 4


def _tc_kernel(x_ref, t_ref, out_ref, acc_ref):
    i = pl.program_id(0)

    @pl.when(i == 0)
    def _init():
        acc_ref[0] = 0.0
        acc_ref[1] = 0.0

    x = x_ref[...]
    m = jnp.max(x, axis=1, keepdims=True)
    e = jnp.exp(x - m)
    ones = jnp.ones((N_COLS, 128), jnp.float32)
    s = lax.dot_general(e, ones, (((1,), (0,)), ((), ())),
                        preferred_element_type=jnp.float32)[:, :1]
    conf = 1.0 / s[:, 0]

    idx = lax.broadcasted_iota(jnp.int32, x.shape, 1)
    pred = jnp.min(jnp.where(x == m, idx, N_COLS), axis=1)
    acc = (pred == t_ref[...]).astype(jnp.float32)

    acc_ref[0] += jnp.sum(acc)
    acc_ref[1] += jnp.sum(conf)

    @pl.when(i == pl.num_programs(0) - 1)
    def _fini():
        out_ref[...] = jnp.stack([acc_ref[0], acc_ref[1]]).reshape(1, 2)


def _tc_part(output, target):
    grid = TC_ROWS // TC_BLOCK_ROWS
    return pl.pallas_call(
        _tc_kernel,
        grid=(grid,),
        in_specs=[
            pl.BlockSpec((TC_BLOCK_ROWS, N_COLS), lambda i: (i, 0)),
            pl.BlockSpec((TC_BLOCK_ROWS,), lambda i: (i,)),
        ],
        out_specs=pl.BlockSpec((1, 2), lambda i: (0, 0)),
        out_shape=jax.ShapeDtypeStruct((1, 2), jnp.float32),
        scratch_shapes=[pltpu.SMEM((2,), jnp.float32)],
    )(output, target)


def _sc_body(x_hbm, t_hbm, acc_out, conf_out, buf0, buf1, tbuf, outv, sem0,
             sem1, tsem, osem):
    wid = lax.axis_index("s") * NC + lax.axis_index("c")
    row0 = SC_ROW0 + wid * RPW

    iota = lax.iota(jnp.int32, L)

    tcopy = pltpu.make_async_copy(t_hbm.at[pl.ds(row0, RPW)], tbuf, tsem)
    tcopy.start()

    bufs = (buf0, buf1)
    sems = (sem0, sem1)

    def start(g):
        cp = pltpu.make_async_copy(
            x_hbm.at[pl.ds(row0 + g * L, L)], bufs[g % 2], sems[g % 2])
        cp.start()
        return cp

    pending = start(0)
    tcopy.wait()

    conf_acc = jnp.zeros((L,), jnp.float32)
    acc_acc = jnp.zeros((L,), jnp.float32)

    for g in range(G):
        pending.wait()
        if g + 1 < G:
            pending = start(g + 1)
        cur = bufs[g % 2]

        def p1(k, ms):
            out = []
            for c in range(UNROLL):
                col = k * UNROLL + c
                colv = jnp.full((L,), col, jnp.int32)
                v = plsc.load_gather(cur, [iota, colv])
                out.append(jnp.maximum(ms[c], v))
            return tuple(out)

        ms = lax.fori_loop(0, N_COLS // UNROLL, p1,
                           tuple(jnp.full((L,), -jnp.inf, jnp.float32)
                                 for _ in range(UNROLL)))
        m = jnp.maximum(jnp.maximum(ms[0], ms[1]), jnp.maximum(ms[2], ms[3]))

        def p2(k, carry):
            ss, ii = carry
            ss, ii = list(ss), list(ii)
            for c in range(UNROLL):
                col = k * UNROLL + c
                colv = jnp.full((L,), col, jnp.int32)
                v = plsc.load_gather(cur, [iota, colv])
                ss[c] = ss[c] + jnp.exp(v - m)
                ii[c] = jnp.minimum(ii[c], jnp.where(v == m, colv, N_COLS))
            return tuple(ss), tuple(ii)

        (ss, ii) = lax.fori_loop(
            0, N_COLS // UNROLL, p2,
            (tuple(jnp.zeros((L,), jnp.float32) for _ in range(UNROLL)),
             tuple(jnp.full((L,), N_COLS, jnp.int32) for _ in range(UNROLL))))
        s = (ss[0] + ss[1]) + (ss[2] + ss[3])
        pred = jnp.minimum(jnp.minimum(ii[0], ii[1]),
                           jnp.minimum(ii[2], ii[3]))

        t_vec = tbuf[pl.ds(g * L, L)]
        conf_acc = conf_acc + 1.0 / s
        acc_acc = acc_acc + jnp.where(pred == t_vec, 1.0, 0.0)

    outv[pl.ds(0, L)] = acc_acc
    outv[pl.ds(L, L)] = conf_acc
    pltpu.make_async_copy(outv.at[pl.ds(0, L)], acc_out.at[wid], osem).start()
    pltpu.make_async_copy(outv.at[pl.ds(0, L)], acc_out.at[wid], osem).wait()
    pltpu.make_async_copy(outv.at[pl.ds(L, L)], conf_out.at[wid], osem).start()
    pltpu.make_async_copy(outv.at[pl.ds(L, L)], conf_out.at[wid], osem).wait()


def _sc_part(output, target):
    mesh = plsc.VectorSubcoreMesh(core_axis_name="c", subcore_axis_name="s")
    f = pl.kernel(
        _sc_body,
        out_type=[
            jax.ShapeDtypeStruct((NW, L), jnp.float32),
            jax.ShapeDtypeStruct((NW, L), jnp.float32),
        ],
        mesh=mesh,
        scratch_types=[
            pltpu.VMEM((L, N_COLS), jnp.float32),
            pltpu.VMEM((L, N_COLS), jnp.float32),
            pltpu.VMEM((RPW,), jnp.int32),
            pltpu.VMEM((2 * L,), jnp.float32),
            pltpu.SemaphoreType.DMA,
            pltpu.SemaphoreType.DMA,
            pltpu.SemaphoreType.DMA,
            pltpu.SemaphoreType.DMA,
        ],
        compiler_params=pltpu.CompilerParams(use_tc_tiling_on_sc=False, needs_layout_passes=False),
    )
    return f(output, target)


@jax.jit
def _ece(output, target):
    target = target.astype(jnp.int32)
    acc_sum = jnp.float32(0)
    conf_sum = jnp.float32(0)
    if TC_ROWS > 0:
        tc = _tc_part(output, target)
        acc_sum += tc[0, 0]
        conf_sum += tc[0, 1]
    if SC_ROWS > 0:
        acc_p, conf_p = _sc_part(output, target)
        acc_sum += jnp.sum(acc_p)
        conf_sum += jnp.sum(conf_p)
    return jnp.abs(acc_sum - conf_sum) / N_ROWS


def kernel(output, target):
    return _ece(output, target)


# speedup vs baseline: 8.7774x; 6.2441x over previous
import functools

import jax
import jax.numpy as jnp
from jax import lax
from jax.experimental import pallas as pl
from jax.experimental.pallas import tpu as pltpu
from jax.experimental.pallas import tpu_sc as plsc

N_ROWS = 16384
N_COLS = 1000

TC_ROWS = 16384
SC_ROWS = N_ROWS - TC_ROWS

TC_BLOCK_ROWS = 2048

NC, NS, L = 2, 16, 16
NW = NC * NS
SC_ROW0 = TC_ROWS
RPW = SC_ROWS // NW
G = RPW // L
UNROLL = 4


def _tc_kernel(x_ref, t_ref, out_ref, acc_ref):
    i = pl.program_id(0)

    @pl.when(i == 0)
    def _init():
        acc_ref[0] = 0.0
        acc_ref[1] = 0.0

    x = x_ref[...]
    m = jnp.max(x, axis=0, keepdims=True)
    e = jnp.exp(x - m)
    ones = jnp.ones((8, N_COLS), jnp.float32)
    s = lax.dot_general(ones, e, (((1,), (0,)), ((), ())),
                        preferred_element_type=jnp.float32)
    conf = 1.0 / s[0, :]

    idx = lax.broadcasted_iota(jnp.int32, x.shape, 0)
    pred = jnp.min(jnp.where(x == m, idx, N_COLS), axis=0)
    acc = (pred == t_ref[...]).astype(jnp.float32)

    acc_ref[0] += jnp.sum(acc)
    acc_ref[1] += jnp.sum(conf)

    @pl.when(i == pl.num_programs(0) - 1)
    def _fini():
        out_ref[...] = jnp.stack([acc_ref[0], acc_ref[1]]).reshape(1, 2)


def _tc_part(output_t, target):
    grid = TC_ROWS // TC_BLOCK_ROWS
    return pl.pallas_call(
        _tc_kernel,
        grid=(grid,),
        in_specs=[
            pl.BlockSpec((N_COLS, TC_BLOCK_ROWS), lambda i: (0, i)),
            pl.BlockSpec((TC_BLOCK_ROWS,), lambda i: (i,)),
        ],
        out_specs=pl.BlockSpec((1, 2), lambda i: (0, 0)),
        out_shape=jax.ShapeDtypeStruct((1, 2), jnp.float32),
        scratch_shapes=[pltpu.SMEM((2,), jnp.float32)],
    )(output_t, target)


def _sc_body(x_hbm, t_hbm, acc_out, conf_out, buf0, buf1, tbuf, outv, sem0,
             sem1, tsem, osem):
    wid = lax.axis_index("s") * NC + lax.axis_index("c")
    row0 = SC_ROW0 + wid * RPW

    iota = lax.iota(jnp.int32, L)

    tcopy = pltpu.make_async_copy(t_hbm.at[pl.ds(row0, RPW)], tbuf, tsem)
    tcopy.start()

    bufs = (buf0, buf1)
    sems = (sem0, sem1)

    def start(g):
        cp = pltpu.make_async_copy(
            x_hbm.at[pl.ds(row0 + g * L, L)], bufs[g % 2], sems[g % 2])
        cp.start()
        return cp

    pending = start(0)
    tcopy.wait()

    conf_acc = jnp.zeros((L,), jnp.float32)
    acc_acc = jnp.zeros((L,), jnp.float32)

    for g in range(G):
        pending.wait()
        if g + 1 < G:
            pending = start(g + 1)
        cur = bufs[g % 2]

        def p1(k, ms):
            out = []
            for c in range(UNROLL):
                col = k * UNROLL + c
                colv = jnp.full((L,), col, jnp.int32)
                v = plsc.load_gather(cur, [iota, colv])
                out.append(jnp.maximum(ms[c], v))
            return tuple(out)

        ms = lax.fori_loop(0, N_COLS // UNROLL, p1,
                           tuple(jnp.full((L,), -jnp.inf, jnp.float32)
                                 for _ in range(UNROLL)))
        m = jnp.maximum(jnp.maximum(ms[0], ms[1]), jnp.maximum(ms[2], ms[3]))

        def p2(k, carry):
            ss, ii = carry
            ss, ii = list(ss), list(ii)
            for c in range(UNROLL):
                col = k * UNROLL + c
                colv = jnp.full((L,), col, jnp.int32)
                v = plsc.load_gather(cur, [iota, colv])
                ss[c] = ss[c] + jnp.exp(v - m)
                ii[c] = jnp.minimum(ii[c], jnp.where(v == m, colv, N_COLS))
            return tuple(ss), tuple(ii)

        (ss, ii) = lax.fori_loop(
            0, N_COLS // UNROLL, p2,
            (tuple(jnp.zeros((L,), jnp.float32) for _ in range(UNROLL)),
             tuple(jnp.full((L,), N_COLS, jnp.int32) for _ in range(UNROLL))))
        s = (ss[0] + ss[1]) + (ss[2] + ss[3])
        pred = jnp.minimum(jnp.minimum(ii[0], ii[1]),
                           jnp.minimum(ii[2], ii[3]))

        t_vec = tbuf[pl.ds(g * L, L)]
        conf_acc = conf_acc + 1.0 / s
        acc_acc = acc_acc + jnp.where(pred == t_vec, 1.0, 0.0)

    outv[pl.ds(0, L)] = acc_acc
    outv[pl.ds(L, L)] = conf_acc
    pltpu.make_async_copy(outv.at[pl.ds(0, L)], acc_out.at[wid], osem).start()
    pltpu.make_async_copy(outv.at[pl.ds(0, L)], acc_out.at[wid], osem).wait()
    pltpu.make_async_copy(outv.at[pl.ds(L, L)], conf_out.at[wid], osem).start()
    pltpu.make_async_copy(outv.at[pl.ds(L, L)], conf_out.at[wid], osem).wait()


def _sc_part(output, target):
    mesh = plsc.VectorSubcoreMesh(core_axis_name="c", subcore_axis_name="s")
    f = pl.kernel(
        _sc_body,
        out_type=[
            jax.ShapeDtypeStruct((NW, L), jnp.float32),
            jax.ShapeDtypeStruct((NW, L), jnp.float32),
        ],
        mesh=mesh,
        scratch_types=[
            pltpu.VMEM((L, N_COLS), jnp.float32),
            pltpu.VMEM((L, N_COLS), jnp.float32),
            pltpu.VMEM((RPW,), jnp.int32),
            pltpu.VMEM((2 * L,), jnp.float32),
            pltpu.SemaphoreType.DMA,
            pltpu.SemaphoreType.DMA,
            pltpu.SemaphoreType.DMA,
            pltpu.SemaphoreType.DMA,
        ],
        compiler_params=pltpu.CompilerParams(use_tc_tiling_on_sc=False, needs_layout_passes=False),
    )
    return f(output, target)


@jax.jit
def _ece(output, target):
    target = target.astype(jnp.int32)
    acc_sum = jnp.float32(0)
    conf_sum = jnp.float32(0)
    if TC_ROWS > 0:
        tc = _tc_part(output.T, target)
        acc_sum += tc[0, 0]
        conf_sum += tc[0, 1]
    if SC_ROWS > 0:
        acc_p, conf_p = _sc_part(output, target)
        acc_sum += jnp.sum(acc_p)
        conf_sum += jnp.sum(conf_p)
    return jnp.abs(acc_sum - conf_sum) / N_ROWS


def kernel(output, target):
    return _ece(output, target)
